# Initial kernel scaffold; baseline (speedup 1.0000x reference)
import functools

import jax
import jax.numpy as jnp
from jax import lax
from jax.experimental import pallas as pl
from jax.experimental.pallas import tpu as pltpu
from jax.experimental.pallas import tpu_sc as plsc

VOCAB = 1000000
HIDDEN = 64
B = 4096
L = 200
N = B * L

NC, NS, LANES = 2, 16, 16
NW = NC * NS
IDX_W = 128
SUB = 4
CHUNK = SUB * IDX_W
PER_W = N // NW
ROWS_PER_W = PER_W // IDX_W
NITER = PER_W // CHUNK


def _fuse_body(dow_ref, hod_ref, out_ref):
    for d in range(8):
        out_ref[d * 25:(d + 1) * 25, :] = dow_ref[d:d + 1, :] + hod_ref[...]


def _fuse_tables(dow_emb, hod_emb):
    return pl.pallas_call(
        _fuse_body,
        out_shape=jax.ShapeDtypeStruct((200, HIDDEN), jnp.float32),
    )(dow_emb, hod_emb)


def _sc_body(enc_hbm, dow_hbm, hod_hbm, emb_hbm, fused_hbm, out_hbm,
             idx_a, idx_b, dv, rows_a, rows_b, sem):
    wid = lax.axis_index("s") * NC + lax.axis_index("c")
    row0 = wid * ROWS_PER_W

    def chunk_body(t, carry):
        rbase = row0 + t * SUB
        obase = (row0 + t * SUB) * IDX_W

        pltpu.sync_copy(enc_hbm.at[pl.ds(rbase, SUB)], idx_a)
        pltpu.sync_copy(dow_hbm.at[pl.ds(rbase, SUB)], dv)
        pltpu.sync_copy(hod_hbm.at[pl.ds(rbase, SUB)], idx_b)

        for j in range(SUB):
            for i in range(IDX_W // LANES):
                sl = pl.ds(i * LANES, LANES)
                idx_b[j, sl] = dv[j, sl] * 25 + idx_b[j, sl]

        copies = []
        for j in range(SUB):
            dst = pl.ds(j * IDX_W, IDX_W)
            copies.append(pltpu.async_copy(
                emb_hbm.at[idx_a.at[j]], rows_a.at[dst], sem))
            copies.append(pltpu.async_copy(
                fused_hbm.at[idx_b.at[j]], rows_b.at[dst], sem))
        for cp in copies:
            cp.wait()

        def add_body(i, c):
            for k in range(HIDDEN // LANES):
                sl = pl.ds(k * LANES, LANES)
                rows_a[i, sl] = rows_a[i, sl] + rows_b[i, sl]
            return c

        lax.fori_loop(0, CHUNK, add_body, 0)

        pltpu.sync_copy(rows_a, out_hbm.at[pl.ds(obase, CHUNK)])
        return carry

    lax.fori_loop(0, NITER, chunk_body, 0)


def _sc_lookup(enc2, dow2, hod2, embedding, fused):
    mesh = plsc.VectorSubcoreMesh(core_axis_name="c", subcore_axis_name="s")
    k = pl.kernel(
        _sc_body,
        out_type=jax.ShapeDtypeStruct((N, HIDDEN), jnp.float32),
        mesh=mesh,
        scratch_types=[
            pltpu.VMEM((SUB, IDX_W), jnp.int32),
            pltpu.VMEM((SUB, IDX_W), jnp.int32),
            pltpu.VMEM((SUB, IDX_W), jnp.int32),
            pltpu.VMEM((CHUNK, HIDDEN), jnp.float32),
            pltpu.VMEM((CHUNK, HIDDEN), jnp.float32),
            pltpu.SemaphoreType.DMA,
        ],
    )
    return k(enc2, dow2, hod2, embedding, fused)


def kernel(enc_src, DoW, HoD, embedding, DoW_Emb, HoD_Emb):
    enc2 = jnp.asarray(enc_src, jnp.int32).reshape(N // IDX_W, IDX_W)
    dow2 = jnp.asarray(DoW, jnp.int32).reshape(N // IDX_W, IDX_W)
    hod2 = jnp.asarray(HoD, jnp.int32).reshape(N // IDX_W, IDX_W)
    fused = _fuse_tables(DoW_Emb.astype(jnp.float32), HoD_Emb.astype(jnp.float32))
    out = _sc_lookup(enc2, dow2, hod2, embedding.astype(jnp.float32), fused)
    return out.reshape(B, L, HIDDEN)


# speedup vs baseline: 3.7799x; 3.7799x over previous
import functools

import jax
import jax.numpy as jnp
from jax import lax
from jax.experimental import pallas as pl
from jax.experimental.pallas import tpu as pltpu
from jax.experimental.pallas import tpu_sc as plsc

VOCAB = 1000000
HIDDEN = 64
B = 4096
L = 200
N = B * L

NC, NS, LANES = 2, 16, 16
NW = NC * NS
IDX_W = 128
SUB = 4
CHUNK = SUB * IDX_W
PER_W = N // NW
ROWS_PER_W = PER_W // IDX_W
NITER = PER_W // CHUNK


def _fuse_body(dow_ref, hod_ref, out_ref):
    for d in range(8):
        out_ref[d * 25:(d + 1) * 25, :] = dow_ref[d:d + 1, :] + hod_ref[...]


def _fuse_tables(dow_emb, hod_emb):
    return pl.pallas_call(
        _fuse_body,
        out_shape=jax.ShapeDtypeStruct((200, HIDDEN), jnp.float32),
    )(dow_emb, hod_emb)


def _sc_body(enc_hbm, dow_hbm, hod_hbm, emb_hbm, fused_hbm, out_hbm,
             idx_a, idx_b, dv, rows_a, rows_b, sem):
    wid = lax.axis_index("s") * NC + lax.axis_index("c")
    row0 = wid * ROWS_PER_W

    def chunk_body(t, carry):
        rbase = row0 + t * SUB
        obase = (row0 + t * SUB) * IDX_W

        pltpu.sync_copy(enc_hbm.at[pl.ds(rbase, SUB)], idx_a)
        pltpu.sync_copy(dow_hbm.at[pl.ds(rbase, SUB)], dv)
        pltpu.sync_copy(hod_hbm.at[pl.ds(rbase, SUB)], idx_b)

        for j in range(SUB):
            for i in range(IDX_W // LANES):
                sl = pl.ds(i * LANES, LANES)
                idx_b[j, sl] = dv[j, sl] * 25 + idx_b[j, sl]

        copies = []
        for j in range(SUB):
            dst = pl.ds(j * IDX_W, IDX_W)
            copies.append(pltpu.async_copy(
                emb_hbm.at[idx_a.at[j]], rows_a.at[dst], sem))
            copies.append(pltpu.async_copy(
                fused_hbm.at[idx_b.at[j]], rows_b.at[dst], sem))
        for cp in copies:
            cp.wait()

        def add_body(i, c):
            for k in range(HIDDEN // LANES):
                sl = pl.ds(k * LANES, LANES)
                rows_a[i, sl] = rows_a[i, sl] + rows_b[i, sl]
            return c

        lax.fori_loop(0, CHUNK, add_body, 0)

        pltpu.sync_copy(rows_a, out_hbm.at[pl.ds(obase, CHUNK)])
        return carry

    lax.fori_loop(0, NITER, chunk_body, 0)


def _sc_lookup(enc2, dow2, hod2, embedding, fused):
    mesh = plsc.VectorSubcoreMesh(core_axis_name="c", subcore_axis_name="s")
    k = pl.kernel(
        _sc_body,
        out_type=jax.ShapeDtypeStruct((N, HIDDEN), jnp.float32),
        mesh=mesh,
        compiler_params=pltpu.CompilerParams(use_tc_tiling_on_sc=False),
        scratch_types=[
            pltpu.VMEM((SUB, IDX_W), jnp.int32),
            pltpu.VMEM((SUB, IDX_W), jnp.int32),
            pltpu.VMEM((SUB, IDX_W), jnp.int32),
            pltpu.VMEM((CHUNK, HIDDEN), jnp.float32),
            pltpu.VMEM((CHUNK, HIDDEN), jnp.float32),
            pltpu.SemaphoreType.DMA,
        ],
    )
    return k(enc2, dow2, hod2, embedding, fused)


def kernel(enc_src, DoW, HoD, embedding, DoW_Emb, HoD_Emb):
    enc2 = jnp.asarray(enc_src, jnp.int32).reshape(N // IDX_W, IDX_W)
    dow2 = jnp.asarray(DoW, jnp.int32).reshape(N // IDX_W, IDX_W)
    hod2 = jnp.asarray(HoD, jnp.int32).reshape(N // IDX_W, IDX_W)
    fused = _fuse_tables(DoW_Emb.astype(jnp.float32), HoD_Emb.astype(jnp.float32))
    out = _sc_lookup(enc2, dow2, hod2, embedding.astype(jnp.float32), fused)
    return out.reshape(B, L, HIDDEN)


# speedup vs baseline: 3.9094x; 1.0343x over previous
import functools

import jax
import jax.numpy as jnp
from jax import lax
from jax.experimental import pallas as pl
from jax.experimental.pallas import tpu as pltpu
from jax.experimental.pallas import tpu_sc as plsc

VOCAB = 1000000
HIDDEN = 64
B = 4096
L = 200
N = B * L

NC, NS, LANES = 2, 16, 16
NW = NC * NS
IDX_W = 128
SUB = 4
CHUNK = SUB * IDX_W
PER_W = N // NW
ROWS_PER_W = PER_W // IDX_W
NITER = PER_W // CHUNK


def _fuse_body(dow_ref, hod_ref, out_ref):
    for d in range(8):
        out_ref[d * 25:(d + 1) * 25, :] = dow_ref[d:d + 1, :] + hod_ref[...]


def _fuse_tables(dow_emb, hod_emb):
    return pl.pallas_call(
        _fuse_body,
        out_shape=jax.ShapeDtypeStruct((200, HIDDEN), jnp.float32),
    )(dow_emb, hod_emb)


def _sc_body(enc_hbm, dow_hbm, hod_hbm, emb_hbm, fused_hbm, out_hbm,
             idx_a, idx_b, dv, rows_a, sem, semi):
    wid = lax.axis_index("s") * NC + lax.axis_index("c")
    row0 = wid * ROWS_PER_W

    def chunk_body(t, carry):
        rbase = row0 + t * SUB
        obase = (row0 + t * SUB) * IDX_W

        c1 = pltpu.async_copy(enc_hbm.at[pl.ds(rbase, SUB)], idx_a, semi)
        c2 = pltpu.async_copy(dow_hbm.at[pl.ds(rbase, SUB)], dv, semi)
        c3 = pltpu.async_copy(hod_hbm.at[pl.ds(rbase, SUB)], idx_b, semi)
        c1.wait(); c2.wait(); c3.wait()

        for j in range(SUB):
            for i in range(IDX_W // LANES):
                sl = pl.ds(i * LANES, LANES)
                idx_b[j, sl] = dv[j, sl] * 25 + idx_b[j, sl]

        copies = []
        for j in range(SUB):
            dst = pl.ds(j * IDX_W, IDX_W)
            copies.append(pltpu.async_copy(
                emb_hbm.at[idx_a.at[j]], rows_a.at[dst], sem))
        for cp in copies:
            cp.wait()

        copies = []
        for j in range(SUB):
            dst = pl.ds(j * IDX_W, IDX_W)
            copies.append(pltpu.async_copy(
                fused_hbm.at[idx_b.at[j]], rows_a.at[dst], sem, add=True))
        for cp in copies:
            cp.wait()

        pltpu.sync_copy(rows_a, out_hbm.at[pl.ds(obase, CHUNK)])
        return carry

    lax.fori_loop(0, NITER, chunk_body, 0)


def _sc_lookup(enc2, dow2, hod2, embedding, fused):
    mesh = plsc.VectorSubcoreMesh(core_axis_name="c", subcore_axis_name="s")
    k = pl.kernel(
        _sc_body,
        out_type=jax.ShapeDtypeStruct((N, HIDDEN), jnp.float32),
        mesh=mesh,
        compiler_params=pltpu.CompilerParams(use_tc_tiling_on_sc=False),
        scratch_types=[
            pltpu.VMEM((SUB, IDX_W), jnp.int32),
            pltpu.VMEM((SUB, IDX_W), jnp.int32),
            pltpu.VMEM((SUB, IDX_W), jnp.int32),
            pltpu.VMEM((CHUNK, HIDDEN), jnp.float32),
            pltpu.SemaphoreType.DMA,
            pltpu.SemaphoreType.DMA,
        ],
    )
    return k(enc2, dow2, hod2, embedding, fused)


def kernel(enc_src, DoW, HoD, embedding, DoW_Emb, HoD_Emb):
    enc2 = jnp.asarray(enc_src, jnp.int32).reshape(N // IDX_W, IDX_W)
    dow2 = jnp.asarray(DoW, jnp.int32).reshape(N // IDX_W, IDX_W)
    hod2 = jnp.asarray(HoD, jnp.int32).reshape(N // IDX_W, IDX_W)
    fused = _fuse_tables(DoW_Emb.astype(jnp.float32), HoD_Emb.astype(jnp.float32))
    out = _sc_lookup(enc2, dow2, hod2, embedding.astype(jnp.float32), fused)
    return out.reshape(B, L, HIDDEN)
